# baseline (speedup 1.0000x reference)
import functools

import jax
import jax.numpy as jnp
from jax import lax
from jax.experimental import pallas as pl
from jax.experimental.pallas import tpu as pltpu
from jax.experimental.pallas import tpu_sc as plsc

_NC = 2
_NS = 16
_NW = _NC * _NS
_IDX_CHUNK = 128


@functools.lru_cache(maxsize=None)
def _build_gather(n, d_item, d_small):
    bpw = n // _NW
    n_chunks = bpw // _IDX_CHUNK
    mesh = plsc.VectorSubcoreMesh(core_axis_name="c", subcore_axis_name="s")

    @functools.partial(
        pl.kernel,
        mesh=mesh,
        compiler_params=pltpu.CompilerParams(use_tc_tiling_on_sc=False),
        out_type=(
            jax.ShapeDtypeStruct((n, d_item), jnp.float32),
            jax.ShapeDtypeStruct((n, d_small), jnp.float32),
            jax.ShapeDtypeStruct((n, d_small), jnp.float32),
        ),
        scratch_types=[
            pltpu.VMEM((bpw,), jnp.int32),
            pltpu.VMEM((bpw,), jnp.int32),
            pltpu.VMEM((bpw,), jnp.int32),
            pltpu.VMEM((bpw, d_item), jnp.float32),
            pltpu.VMEM((bpw, d_small), jnp.float32),
            pltpu.VMEM((bpw, d_small), jnp.float32),
            pltpu.SemaphoreType.DMA,
        ],
    )
    def gather3(item_ids, key_ids, genre_ids, item_emb, key_emb, genre_emb,
                item_out, key_out, genre_out,
                idx_i, idx_k, idx_g, rows_i, rows_k, rows_g, sem):
        wid = lax.axis_index("s") * _NC + lax.axis_index("c")
        base = wid * bpw
        pltpu.sync_copy(item_ids.at[pl.ds(base, bpw)], idx_i)
        pltpu.sync_copy(key_ids.at[pl.ds(base, bpw)], idx_k)
        pltpu.sync_copy(genre_ids.at[pl.ds(base, bpw)], idx_g)
        copies = []
        for j in range(n_chunks):
            sl = pl.ds(j * _IDX_CHUNK, _IDX_CHUNK)
            copies.append(pltpu.async_copy(
                item_emb.at[idx_i.at[sl]], rows_i.at[sl], sem))
            copies.append(pltpu.async_copy(
                key_emb.at[idx_k.at[sl]], rows_k.at[sl], sem))
            copies.append(pltpu.async_copy(
                genre_emb.at[idx_g.at[sl]], rows_g.at[sl], sem))
        for c in copies:
            c.wait()
        pltpu.sync_copy(rows_i, item_out.at[pl.ds(base, bpw)])
        pltpu.sync_copy(rows_k, key_out.at[pl.ds(base, bpw)])
        pltpu.sync_copy(rows_g, genre_out.at[pl.ds(base, bpw)])

    return gather3


def _gelu(x):
    return 0.5 * x * (1.0 + lax.erf(x * 0.7071067811865476))


def _dot(a, b):
    return jnp.dot(a, b, precision=lax.Precision.HIGHEST,
                   preferred_element_type=jnp.float32)


def _mlp_body(items, keys, genres, audio, wa, ba,
              w1i, w1k, w1g, w1a, b1, w2, b2, out):
    a = _gelu(_dot(audio[...], wa[...]) + ba[...])
    h = _dot(items[...], w1i[...])
    h = h + _dot(keys[...], w1k[...])
    h = h + _dot(genres[...], w1g[...])
    h = h + _dot(a, w1a[...])
    h = _gelu(h + b1[...])
    out[...] = _dot(h, w2[...]) + b2[...]


def kernel(item_ids, key_ids, genre_ids, audio_cont, item_emb, key_emb,
           genre_emb, W_audio, b_audio, W1, b1, W2, b2):
    n = item_ids.shape[0]
    d_item = item_emb.shape[1]
    d_small = key_emb.shape[1]
    d_audio = W_audio.shape[1]
    d_hid = W1.shape[1]
    d_out = W2.shape[1]

    gather3 = _build_gather(n, d_item, d_small)
    items, keys, genres = gather3(
        item_ids.astype(jnp.int32), key_ids.astype(jnp.int32),
        genre_ids.astype(jnp.int32), item_emb, key_emb, genre_emb)

    w1i = W1[:d_item]
    w1k = W1[d_item:d_item + d_small]
    w1g = W1[d_item + d_small:d_item + 2 * d_small]
    w1a = W1[d_item + 2 * d_small:]

    bn = min(n, 2048)
    grid = (n // bn,)

    def row_spec(d):
        return pl.BlockSpec((bn, d), lambda i: (i, 0))

    def rep_spec(r, c):
        return pl.BlockSpec((r, c), lambda i: (0, 0))

    return pl.pallas_call(
        _mlp_body,
        grid=grid,
        in_specs=[
            row_spec(d_item), row_spec(d_small), row_spec(d_small),
            row_spec(audio_cont.shape[1]),
            rep_spec(W_audio.shape[0], d_audio), rep_spec(1, d_audio),
            rep_spec(d_item, d_hid), rep_spec(d_small, d_hid),
            rep_spec(d_small, d_hid), rep_spec(d_audio, d_hid),
            rep_spec(1, d_hid),
            rep_spec(d_hid, d_out), rep_spec(1, d_out),
        ],
        out_specs=row_spec(d_out),
        out_shape=jax.ShapeDtypeStruct((n, d_out), jnp.float32),
    )(items, keys, genres, audio_cont,
      W_audio, b_audio.reshape(1, -1),
      w1i, w1k, w1g, w1a, b1.reshape(1, -1),
      W2, b2.reshape(1, -1))
